# baseline (speedup 1.0000x reference)
import jax
import jax.numpy as jnp
from jax.experimental import pallas as pl
from jax.experimental.pallas import tpu as pltpu


def _k1_body(x_ref, w_ref, b_ref, o_ref):
    x = x_ref[0].astype(jnp.bfloat16)
    y = jax.lax.dot_general(x, w_ref[...], (((0,), (0,)), ((), ())),
                            preferred_element_type=jnp.float32)
    o_ref[0] = (y + b_ref[...]).astype(jnp.bfloat16)


def _make_k1_add_body(W):
    def _body(x_ref, w_ref, b_ref, s_ref, o_ref):
        x = x_ref[0].astype(jnp.bfloat16)
        y = jax.lax.dot_general(x, w_ref[...], (((0,), (0,)), ((), ())),
                                preferred_element_type=jnp.float32)
        TS, C = y.shape
        src = s_ref[0].astype(jnp.float32)
        up = jnp.repeat(src, 2, axis=0)
        up = jnp.repeat(up.reshape(TS // (2 * W), W, C), 2, axis=0)
        o_ref[0] = (y + b_ref[...] + up.reshape(TS, C)).astype(jnp.bfloat16)
    return _body


def _k1_tile(H, W, has_add, target=1024):
    if not has_add:
        return min(H * W, target)
    k = 1
    while 4 * k * W <= target and H % (4 * k) == 0:
        k *= 2
    return 2 * k * W


def _lateral(feat, w_oihw, bias, src_flat, src_hw):
    N, Cin, H, W = feat.shape
    C = w_oihw.shape[0]
    HW = H * W
    x = feat.reshape(N, Cin, HW)
    w2 = jnp.transpose(w_oihw[:, :, 0, 0], (1, 0)).astype(jnp.bfloat16)
    b2 = bias.reshape(1, C).astype(jnp.float32)

    TS = _k1_tile(H, W, src_flat is not None)
    grid = (N, HW // TS)
    in_specs = [
        pl.BlockSpec((1, Cin, TS), lambda n, j: (n, 0, j)),
        pl.BlockSpec((Cin, C), lambda n, j: (0, 0)),
        pl.BlockSpec((1, C), lambda n, j: (0, 0)),
    ]
    args = [x, w2, b2]
    if src_flat is None:
        body = _k1_body
    else:
        body = _make_k1_add_body(W)
        in_specs.append(pl.BlockSpec((1, TS // 4, C), lambda n, j: (n, j, 0)))
        args.append(src_flat)

    bytes_acc = (N * HW * Cin * 4 + Cin * C * 2 + C * 4 + N * HW * C * 2
                 + (0 if src_flat is None else N * HW // 4 * C * 2))
    out = pl.pallas_call(
        body,
        out_shape=jax.ShapeDtypeStruct((N, HW, C), jnp.bfloat16),
        grid=grid,
        in_specs=in_specs,
        out_specs=pl.BlockSpec((1, TS, C), lambda n, j: (n, j, 0)),
        compiler_params=pltpu.CompilerParams(
            dimension_semantics=("parallel", "parallel"),
            vmem_limit_bytes=64 * 1024 * 1024,
        ),
        cost_estimate=pl.CostEstimate(
            flops=int(2 * N * HW * Cin * C), transcendentals=0,
            bytes_accessed=int(bytes_acc)),
    )(*args)
    return out


def _make_k2_body(TH, W, Ht):
    def _body(xp_ref, xc_ref, xn_ref, w_ref, b_ref, o_ref):
        i = pl.program_id(1)
        C = xc_ref.shape[-1]
        top = jnp.where(i > 0, xp_ref[0, TH - 1:TH], 0.0)
        bot = jnp.where(i < Ht - 1, xn_ref[0, 0:1], 0.0)
        xw = jnp.concatenate([top, xc_ref[0], bot], axis=0)
        zcol = jnp.zeros((TH + 2, 1, C), xw.dtype)
        wp = jnp.concatenate([zcol, xw, zcol], axis=1)
        xcat = jnp.concatenate([wp[:, dx:dx + W, :] for dx in range(3)],
                               axis=-1)
        b = b_ref[...]
        rows = []
        for t in range(TH):
            acc = jnp.dot(xcat[t], w_ref[0],
                          preferred_element_type=jnp.float32)
            acc += jnp.dot(xcat[t + 1], w_ref[1],
                           preferred_element_type=jnp.float32)
            acc += jnp.dot(xcat[t + 2], w_ref[2],
                           preferred_element_type=jnp.float32)
            rows.append(jnp.transpose(acc + b, (1, 0)))
        o_ref[0] = jnp.stack(rows, axis=1)
    return _body


def _k2_row_tile(H):
    d = 1
    for th in range(1, min(16, H) + 1):
        if H % th == 0:
            d = th
    if d == H and d % 2 == 0 and H > 2:
        d //= 2
    return d


def _smooth(inner_flat, w_oihw, bias, N, H, W):
    C = w_oihw.shape[0]
    x = inner_flat.reshape(N, H, W, C)
    TH = _k2_row_tile(H)
    Ht = H // TH
    w3 = jnp.transpose(w_oihw, (2, 3, 1, 0)).reshape(3, 3 * C, C)
    w3 = w3.astype(jnp.bfloat16)
    b2 = bias.reshape(1, C).astype(jnp.float32)

    def _clamped(k):
        return lambda n, i: (n, jnp.clip(i + k, 0, Ht - 1), 0, 0)

    in_specs = [pl.BlockSpec((1, TH, W, C), _clamped(k)) for k in (-1, 0, 1)]
    in_specs += [
        pl.BlockSpec((3, 3 * C, C), lambda n, i: (0, 0, 0)),
        pl.BlockSpec((1, C), lambda n, i: (0, 0)),
    ]
    flops = 2 * N * H * W * 9 * C * C
    bytes_acc = (3 * N * H * W * C * 2 + 9 * C * C * 2 + C * 4
                 + N * H * W * C * 4)
    out = pl.pallas_call(
        _make_k2_body(TH, W, Ht),
        out_shape=jax.ShapeDtypeStruct((N, C, H, W), jnp.float32),
        grid=(N, Ht),
        in_specs=in_specs,
        out_specs=pl.BlockSpec((1, C, TH, W), lambda n, i: (n, 0, i, 0)),
        compiler_params=pltpu.CompilerParams(
            dimension_semantics=("parallel", "parallel"),
            vmem_limit_bytes=64 * 1024 * 1024,
        ),
        cost_estimate=pl.CostEstimate(
            flops=int(flops), transcendentals=0,
            bytes_accessed=int(bytes_acc)),
    )(x, x, x, w3, b2)
    return out


def kernel(feat0, feat1, feat2, feat3,
           inner_w0, inner_b0, layer_w0, layer_b0,
           inner_w1, inner_b1, layer_w1, layer_b1,
           inner_w2, inner_b2, layer_w2, layer_b2,
           inner_w3, inner_b3, layer_w3, layer_b3):
    feats = [feat0, feat1, feat2, feat3]
    iw = [inner_w0, inner_w1, inner_w2, inner_w3]
    ib = [inner_b0, inner_b1, inner_b2, inner_b3]
    lw = [layer_w0, layer_w1, layer_w2, layer_w3]
    lb = [layer_b0, layer_b1, layer_b2, layer_b3]

    names = ["feat0", "feat1", "feat2", "feat3"]
    results = [None] * 4
    last_inner = None
    last_hw = None
    for idx in range(3, -1, -1):
        N, _, H, W = feats[idx].shape
        last_inner = _lateral(feats[idx], iw[idx], ib[idx],
                              last_inner, last_hw)
        last_hw = (H, W)
        results[idx] = _smooth(last_inner, lw[idx], lb[idx], N, H, W)

    from collections import OrderedDict
    return OrderedDict(zip(names, results))


# speedup vs baseline: 1.0839x; 1.0839x over previous
import jax
import jax.numpy as jnp
from jax.experimental import pallas as pl
from jax.experimental.pallas import tpu as pltpu


def _k1_body(x_ref, w_ref, b_ref, o_ref):
    x = x_ref[0].astype(jnp.bfloat16)
    y = jnp.dot(w_ref[...], x, preferred_element_type=jnp.float32)
    o_ref[0] = (y + b_ref[...]).astype(jnp.bfloat16)


def _k1_add_body(x_ref, w_ref, b_ref, s_ref, g_ref, o_ref):
    x = x_ref[0].astype(jnp.bfloat16)
    y = jnp.dot(w_ref[...], x, preferred_element_type=jnp.float32)
    up = jnp.dot(s_ref[0], g_ref[...],
                 preferred_element_type=jnp.float32)
    o_ref[0] = (y + b_ref[...] + up).astype(jnp.bfloat16)


def _upsample_gather(W, TS):
    jj = jnp.arange(TS)
    src = (jj // (2 * W)) * (W // 2) + (jj % W) // 2
    return (src[None, :] == jnp.arange(TS // 4)[:, None]).astype(jnp.bfloat16)


def _lateral(feat, w_oihw, bias, src_flat):
    N, Cin, H, W = feat.shape
    C = w_oihw.shape[0]
    HW = H * W
    x = feat.reshape(N, Cin, HW)
    w2 = w_oihw[:, :, 0, 0].astype(jnp.bfloat16)
    b2 = bias.reshape(C, 1).astype(jnp.float32)

    TS = HW if src_flat is None else min(512, HW)
    grid = (N, HW // TS)
    in_specs = [
        pl.BlockSpec((1, Cin, TS), lambda n, j: (n, 0, j)),
        pl.BlockSpec((C, Cin), lambda n, j: (0, 0)),
        pl.BlockSpec((C, 1), lambda n, j: (0, 0)),
    ]
    args = [x, w2, b2]
    if src_flat is None:
        body = _k1_body
    else:
        body = _k1_add_body
        in_specs += [
            pl.BlockSpec((1, C, TS // 4), lambda n, j: (n, 0, j)),
            pl.BlockSpec((TS // 4, TS), lambda n, j: (0, 0)),
        ]
        args += [src_flat, _upsample_gather(W, TS)]

    bytes_acc = (N * HW * Cin * 4 + Cin * C * 2 + C * 4 + N * HW * C * 2
                 + (0 if src_flat is None else N * HW // 4 * C * 2))
    out = pl.pallas_call(
        body,
        out_shape=jax.ShapeDtypeStruct((N, C, HW), jnp.bfloat16),
        grid=grid,
        in_specs=in_specs,
        out_specs=pl.BlockSpec((1, C, TS), lambda n, j: (n, 0, j)),
        compiler_params=pltpu.CompilerParams(
            dimension_semantics=("parallel", "parallel"),
            vmem_limit_bytes=64 * 1024 * 1024,
        ),
        cost_estimate=pl.CostEstimate(
            flops=int(2 * N * HW * Cin * C), transcendentals=0,
            bytes_accessed=int(bytes_acc)),
    )(*args)
    return out


def _make_k2_body(TH, W, Ht):
    S = TH * W

    def _body(xp_ref, xc_ref, xn_ref, w_ref, b_ref, o_ref):
        i = pl.program_id(1)
        C = xc_ref.shape[1]
        dt = xc_ref.dtype
        top = jnp.where(i > 0, xp_ref[0, :, S - W:], jnp.zeros((C, W), dt))
        bot = jnp.where(i < Ht - 1, xn_ref[0, :, :W], jnp.zeros((C, W), dt))
        xfull = jnp.concatenate([top, xc_ref[0], bot], axis=1)
        lane = jax.lax.broadcasted_iota(jnp.int32, (1, S + 2 * W), 1) % W
        zc = jnp.zeros((C, 1), dt)
        sL = jnp.concatenate([zc, xfull[:, :-1]], axis=1)
        sL = jnp.where(lane == 0, jnp.zeros((), dt), sL)
        sR = jnp.concatenate([xfull[:, 1:], zc], axis=1)
        sR = jnp.where(lane == W - 1, jnp.zeros((), dt), sR)
        srcs = (sL, xfull, sR)
        acc = None
        for dy in range(3):
            for dx in range(3):
                op = srcs[dx][:, dy * W: dy * W + S]
                d = jnp.dot(w_ref[3 * dy + dx], op,
                            preferred_element_type=jnp.float32)
                acc = d if acc is None else acc + d
        o_ref[0] = acc + b_ref[...]
    return _body


def _k2_row_tile(H, W):
    best = 1
    for th in range(1, H + 1):
        if H % th == 0 and th * W <= 2048 and H // th >= 2:
            best = th
    return best


def _smooth(inner_flat, w_oihw, bias, N, H, W):
    C = w_oihw.shape[0]
    TH = _k2_row_tile(H, W)
    Ht = H // TH
    w9 = jnp.transpose(w_oihw, (2, 3, 0, 1)).reshape(9, C, C)
    w9 = w9.astype(jnp.bfloat16)
    b2 = bias.reshape(C, 1).astype(jnp.float32)

    def _clamped(k):
        return lambda n, i: (n, 0, jnp.clip(i + k, 0, Ht - 1))

    in_specs = [pl.BlockSpec((1, C, TH * W), _clamped(k)) for k in (-1, 0, 1)]
    in_specs += [
        pl.BlockSpec((9, C, C), lambda n, i: (0, 0, 0)),
        pl.BlockSpec((C, 1), lambda n, i: (0, 0)),
    ]
    flops = 2 * N * H * W * 9 * C * C
    bytes_acc = (3 * N * H * W * C * 2 + 9 * C * C * 2 + C * 4
                 + N * H * W * C * 4)
    out = pl.pallas_call(
        _make_k2_body(TH, W, Ht),
        out_shape=jax.ShapeDtypeStruct((N, C, H * W), jnp.float32),
        grid=(N, Ht),
        in_specs=in_specs,
        out_specs=pl.BlockSpec((1, C, TH * W), lambda n, i: (n, 0, i)),
        compiler_params=pltpu.CompilerParams(
            dimension_semantics=("parallel", "parallel"),
            vmem_limit_bytes=64 * 1024 * 1024,
        ),
        cost_estimate=pl.CostEstimate(
            flops=int(flops), transcendentals=0,
            bytes_accessed=int(bytes_acc)),
    )(inner_flat, inner_flat, inner_flat, w9, b2)
    return out.reshape(N, C, H, W)


def kernel(feat0, feat1, feat2, feat3,
           inner_w0, inner_b0, layer_w0, layer_b0,
           inner_w1, inner_b1, layer_w1, layer_b1,
           inner_w2, inner_b2, layer_w2, layer_b2,
           inner_w3, inner_b3, layer_w3, layer_b3):
    feats = [feat0, feat1, feat2, feat3]
    iw = [inner_w0, inner_w1, inner_w2, inner_w3]
    ib = [inner_b0, inner_b1, inner_b2, inner_b3]
    lw = [layer_w0, layer_w1, layer_w2, layer_w3]
    lb = [layer_b0, layer_b1, layer_b2, layer_b3]

    names = ["feat0", "feat1", "feat2", "feat3"]
    results = [None] * 4
    last_inner = None
    for idx in range(3, -1, -1):
        N, _, H, W = feats[idx].shape
        last_inner = _lateral(feats[idx], iw[idx], ib[idx], last_inner)
        results[idx] = _smooth(last_inner, lw[idx], lb[idx], N, H, W)

    from collections import OrderedDict
    return OrderedDict(zip(names, results))


# speedup vs baseline: 1.0895x; 1.0052x over previous
import jax
import jax.numpy as jnp
from jax.experimental import pallas as pl
from jax.experimental.pallas import tpu as pltpu


def _k1_body(x_ref, w_ref, b_ref, o_ref):
    x = x_ref[0].astype(jnp.bfloat16)
    y = jnp.dot(w_ref[...], x, preferred_element_type=jnp.float32)
    o_ref[0] = (y + b_ref[...]).astype(jnp.bfloat16)


def _k1_add_body(x_ref, w_ref, b_ref, s_ref, g_ref, o_ref):
    x = x_ref[0].astype(jnp.bfloat16)
    y = jnp.dot(w_ref[...], x, preferred_element_type=jnp.float32)
    up = jnp.dot(s_ref[0], g_ref[...],
                 preferred_element_type=jnp.float32)
    o_ref[0] = (y + b_ref[...] + up).astype(jnp.bfloat16)


def _upsample_gather(W, TS):
    jj = jnp.arange(TS)
    src = (jj // (2 * W)) * (W // 2) + (jj % W) // 2
    return (src[None, :] == jnp.arange(TS // 4)[:, None]).astype(jnp.bfloat16)


def _lateral(feat, w_oihw, bias, src_flat):
    N, Cin, H, W = feat.shape
    C = w_oihw.shape[0]
    HW = H * W
    x = feat.reshape(N, Cin, HW)
    w2 = w_oihw[:, :, 0, 0].astype(jnp.bfloat16)
    b2 = bias.reshape(C, 1).astype(jnp.float32)

    TS = HW if src_flat is None else min(512, HW)
    grid = (N, HW // TS)
    in_specs = [
        pl.BlockSpec((1, Cin, TS), lambda n, j: (n, 0, j)),
        pl.BlockSpec((C, Cin), lambda n, j: (0, 0)),
        pl.BlockSpec((C, 1), lambda n, j: (0, 0)),
    ]
    args = [x, w2, b2]
    if src_flat is None:
        body = _k1_body
    else:
        body = _k1_add_body
        in_specs += [
            pl.BlockSpec((1, C, TS // 4), lambda n, j: (n, 0, j)),
            pl.BlockSpec((TS // 4, TS), lambda n, j: (0, 0)),
        ]
        args += [src_flat, _upsample_gather(W, TS)]

    bytes_acc = (N * HW * Cin * 4 + Cin * C * 2 + C * 4 + N * HW * C * 2
                 + (0 if src_flat is None else N * HW // 4 * C * 2))
    out = pl.pallas_call(
        body,
        out_shape=jax.ShapeDtypeStruct((N, C, HW), jnp.bfloat16),
        grid=grid,
        in_specs=in_specs,
        out_specs=pl.BlockSpec((1, C, TS), lambda n, j: (n, 0, j)),
        compiler_params=pltpu.CompilerParams(
            dimension_semantics=("parallel", "parallel"),
            vmem_limit_bytes=64 * 1024 * 1024,
        ),
        cost_estimate=pl.CostEstimate(
            flops=int(2 * N * HW * Cin * C), transcendentals=0,
            bytes_accessed=int(bytes_acc)),
    )(*args)
    return out


def _make_k2_body(TH, W, Ht, row_halo):
    S = TH * W

    def _body(xp_ref, xc_ref, xn_ref, w_ref, b_ref, o_ref):
        i = pl.program_id(1)
        C = xc_ref.shape[1]
        dt = xc_ref.dtype
        prev = xp_ref[0] if row_halo else xp_ref[0, :, S - W:]
        nxt = xn_ref[0] if row_halo else xn_ref[0, :, :W]
        top = jnp.where(i > 0, prev, jnp.zeros((C, W), dt))
        bot = jnp.where(i < Ht - 1, nxt, jnp.zeros((C, W), dt))
        xfull = jnp.concatenate([top, xc_ref[0], bot], axis=1)
        lane = jax.lax.broadcasted_iota(jnp.int32, (1, S + 2 * W), 1) % W
        zc = jnp.zeros((C, 1), dt)
        sL = jnp.concatenate([zc, xfull[:, :-1]], axis=1)
        sL = jnp.where(lane == 0, jnp.zeros((), dt), sL)
        sR = jnp.concatenate([xfull[:, 1:], zc], axis=1)
        sR = jnp.where(lane == W - 1, jnp.zeros((), dt), sR)
        srcs = (sL, xfull, sR)
        acc = None
        for dy in range(3):
            for dx in range(3):
                op = srcs[dx][:, dy * W: dy * W + S]
                d = jnp.dot(w_ref[3 * dy + dx], op,
                            preferred_element_type=jnp.float32)
                acc = d if acc is None else acc + d
        o_ref[0] = acc + b_ref[...]
    return _body


def _k2_row_tile(H, W):
    best = 1
    for th in range(1, H + 1):
        if H % th == 0 and th * W <= 2048 and H // th >= 2:
            best = th
    return best


def _smooth(inner_flat, w_oihw, bias, N, H, W):
    C = w_oihw.shape[0]
    TH = _k2_row_tile(H, W)
    Ht = H // TH
    w9 = jnp.transpose(w_oihw, (2, 3, 0, 1)).reshape(9, C, C)
    w9 = w9.astype(jnp.bfloat16)
    b2 = bias.reshape(C, 1).astype(jnp.float32)

    row_halo = (W % 128 == 0)
    if row_halo:
        in_specs = [
            pl.BlockSpec((1, C, W),
                         lambda n, i: (n, 0, jnp.clip(i * TH - 1, 0, H - 1))),
            pl.BlockSpec((1, C, TH * W), lambda n, i: (n, 0, i)),
            pl.BlockSpec((1, C, W),
                         lambda n, i: (n, 0, jnp.clip((i + 1) * TH, 0, H - 1))),
        ]
    else:
        in_specs = [
            pl.BlockSpec((1, C, TH * W),
                         lambda n, i, _k=k: (n, 0, jnp.clip(i + _k, 0, Ht - 1)))
            for k in (-1, 0, 1)
        ]
    in_specs += [
        pl.BlockSpec((9, C, C), lambda n, i: (0, 0, 0)),
        pl.BlockSpec((C, 1), lambda n, i: (0, 0)),
    ]
    flops = 2 * N * H * W * 9 * C * C
    bytes_acc = (N * (H + 2 * Ht) * W * C * 2 + 9 * C * C * 2 + C * 4
                 + N * H * W * C * 4)
    out = pl.pallas_call(
        _make_k2_body(TH, W, Ht, row_halo),
        out_shape=jax.ShapeDtypeStruct((N, C, H * W), jnp.float32),
        grid=(N, Ht),
        in_specs=in_specs,
        out_specs=pl.BlockSpec((1, C, TH * W), lambda n, i: (n, 0, i)),
        compiler_params=pltpu.CompilerParams(
            dimension_semantics=("parallel", "parallel"),
            vmem_limit_bytes=64 * 1024 * 1024,
        ),
        cost_estimate=pl.CostEstimate(
            flops=int(flops), transcendentals=0,
            bytes_accessed=int(bytes_acc)),
    )(inner_flat, inner_flat, inner_flat, w9, b2)
    return out.reshape(N, C, H, W)


def kernel(feat0, feat1, feat2, feat3,
           inner_w0, inner_b0, layer_w0, layer_b0,
           inner_w1, inner_b1, layer_w1, layer_b1,
           inner_w2, inner_b2, layer_w2, layer_b2,
           inner_w3, inner_b3, layer_w3, layer_b3):
    feats = [feat0, feat1, feat2, feat3]
    iw = [inner_w0, inner_w1, inner_w2, inner_w3]
    ib = [inner_b0, inner_b1, inner_b2, inner_b3]
    lw = [layer_w0, layer_w1, layer_w2, layer_w3]
    lb = [layer_b0, layer_b1, layer_b2, layer_b3]

    names = ["feat0", "feat1", "feat2", "feat3"]
    results = [None] * 4
    last_inner = None
    for idx in range(3, -1, -1):
        N, _, H, W = feats[idx].shape
        last_inner = _lateral(feats[idx], iw[idx], ib[idx], last_inner)
        results[idx] = _smooth(last_inner, lw[idx], lb[idx], N, H, W)

    from collections import OrderedDict
    return OrderedDict(zip(names, results))


# speedup vs baseline: 1.1789x; 1.0820x over previous
import jax
import jax.numpy as jnp
from jax.experimental import pallas as pl
from jax.experimental.pallas import tpu as pltpu


def _k1_body(x_ref, w_ref, b_ref, o_ref):
    x = x_ref[0].astype(jnp.bfloat16)
    y = jnp.dot(w_ref[...], x, preferred_element_type=jnp.float32)
    o_ref[0] = (y + b_ref[...]).astype(jnp.bfloat16)


def _k1_add_body(x_ref, w_ref, b_ref, s_ref, g_ref, o_ref):
    x = x_ref[0].astype(jnp.bfloat16)
    y = jnp.dot(w_ref[...], x, preferred_element_type=jnp.float32)
    up = jnp.dot(s_ref[0], g_ref[...],
                 preferred_element_type=jnp.float32)
    o_ref[0] = (y + b_ref[...] + up).astype(jnp.bfloat16)


def _upsample_gather(W, TS):
    jj = jnp.arange(TS)
    src = (jj // (2 * W)) * (W // 2) + (jj % W) // 2
    return (src[None, :] == jnp.arange(TS // 4)[:, None]).astype(jnp.bfloat16)


def _lateral(feat, w_oihw, bias, src_flat):
    N, Cin, H, W = feat.shape
    C = w_oihw.shape[0]
    HW = H * W
    x = feat.reshape(N, Cin, HW)
    w2 = w_oihw[:, :, 0, 0].astype(jnp.bfloat16)
    b2 = bias.reshape(C, 1).astype(jnp.float32)

    TS = HW if src_flat is None else min(1024, HW)
    grid = (N, HW // TS)
    in_specs = [
        pl.BlockSpec((1, Cin, TS), lambda n, j: (n, 0, j)),
        pl.BlockSpec((C, Cin), lambda n, j: (0, 0)),
        pl.BlockSpec((C, 1), lambda n, j: (0, 0)),
    ]
    args = [x, w2, b2]
    if src_flat is None:
        body = _k1_body
    else:
        body = _k1_add_body
        in_specs += [
            pl.BlockSpec((1, C, TS // 4), lambda n, j: (n, 0, j)),
            pl.BlockSpec((TS // 4, TS), lambda n, j: (0, 0)),
        ]
        args += [src_flat, _upsample_gather(W, TS)]

    bytes_acc = (N * HW * Cin * 4 + Cin * C * 2 + C * 4 + N * HW * C * 2
                 + (0 if src_flat is None else N * HW // 4 * C * 2))
    out = pl.pallas_call(
        body,
        out_shape=jax.ShapeDtypeStruct((N, C, HW), jnp.bfloat16),
        grid=grid,
        in_specs=in_specs,
        out_specs=pl.BlockSpec((1, C, TS), lambda n, j: (n, 0, j)),
        compiler_params=pltpu.CompilerParams(
            dimension_semantics=("parallel", "parallel"),
            vmem_limit_bytes=64 * 1024 * 1024,
        ),
        cost_estimate=pl.CostEstimate(
            flops=int(2 * N * HW * Cin * C), transcendentals=0,
            bytes_accessed=int(bytes_acc)),
    )(*args)
    return out


def _make_k2_body(TH, W, Ht, row_halo):
    S = TH * W

    def _body(xp_ref, xc_ref, xn_ref, w_ref, b_ref, o_ref):
        i = pl.program_id(1)
        C = xc_ref.shape[1]
        dt = xc_ref.dtype
        prev = xp_ref[0] if row_halo else xp_ref[0, :, S - W:]
        nxt = xn_ref[0] if row_halo else xn_ref[0, :, :W]
        top = jnp.where(i > 0, prev, jnp.zeros((C, W), dt))
        bot = jnp.where(i < Ht - 1, nxt, jnp.zeros((C, W), dt))
        xfull = jnp.concatenate([top, xc_ref[0], bot], axis=1)
        lane = jax.lax.broadcasted_iota(jnp.int32, (1, S + 2 * W), 1) % W
        zc = jnp.zeros((C, 1), dt)
        sL = jnp.concatenate([zc, xfull[:, :-1]], axis=1)
        sL = jnp.where(lane == 0, jnp.zeros((), dt), sL)
        sR = jnp.concatenate([xfull[:, 1:], zc], axis=1)
        sR = jnp.where(lane == W - 1, jnp.zeros((), dt), sR)
        srcs = (sL, xfull, sR)
        acc = None
        for dy in range(3):
            for dx in range(3):
                op = srcs[dx][:, dy * W: dy * W + S]
                d = jnp.dot(w_ref[3 * dy + dx], op,
                            preferred_element_type=jnp.float32)
                acc = d if acc is None else acc + d
        o_ref[0] = acc + b_ref[...]
    return _body


def _k2_row_tile(H, W):
    best = 1
    for th in range(1, H + 1):
        if H % th == 0 and th * W <= 4096 and H // th >= 2:
            best = th
    return best


def _smooth(inner_flat, w_oihw, bias, N, H, W):
    C = w_oihw.shape[0]
    TH = _k2_row_tile(H, W)
    Ht = H // TH
    w9 = jnp.transpose(w_oihw, (2, 3, 0, 1)).reshape(9, C, C)
    w9 = w9.astype(jnp.bfloat16)
    b2 = bias.reshape(C, 1).astype(jnp.float32)

    row_halo = (W % 128 == 0)
    if row_halo:
        in_specs = [
            pl.BlockSpec((1, C, W),
                         lambda n, i: (n, 0, jnp.clip(i * TH - 1, 0, H - 1))),
            pl.BlockSpec((1, C, TH * W), lambda n, i: (n, 0, i)),
            pl.BlockSpec((1, C, W),
                         lambda n, i: (n, 0, jnp.clip((i + 1) * TH, 0, H - 1))),
        ]
    else:
        in_specs = [
            pl.BlockSpec((1, C, TH * W),
                         lambda n, i, _k=k: (n, 0, jnp.clip(i + _k, 0, Ht - 1)))
            for k in (-1, 0, 1)
        ]
    in_specs += [
        pl.BlockSpec((9, C, C), lambda n, i: (0, 0, 0)),
        pl.BlockSpec((C, 1), lambda n, i: (0, 0)),
    ]
    flops = 2 * N * H * W * 9 * C * C
    bytes_acc = (N * (H + 2 * Ht) * W * C * 2 + 9 * C * C * 2 + C * 4
                 + N * H * W * C * 4)
    out = pl.pallas_call(
        _make_k2_body(TH, W, Ht, row_halo),
        out_shape=jax.ShapeDtypeStruct((N, C, H * W), jnp.float32),
        grid=(N, Ht),
        in_specs=in_specs,
        out_specs=pl.BlockSpec((1, C, TH * W), lambda n, i: (n, 0, i)),
        compiler_params=pltpu.CompilerParams(
            dimension_semantics=("parallel", "parallel"),
            vmem_limit_bytes=64 * 1024 * 1024,
        ),
        cost_estimate=pl.CostEstimate(
            flops=int(flops), transcendentals=0,
            bytes_accessed=int(bytes_acc)),
    )(inner_flat, inner_flat, inner_flat, w9, b2)
    return out.reshape(N, C, H, W)


def kernel(feat0, feat1, feat2, feat3,
           inner_w0, inner_b0, layer_w0, layer_b0,
           inner_w1, inner_b1, layer_w1, layer_b1,
           inner_w2, inner_b2, layer_w2, layer_b2,
           inner_w3, inner_b3, layer_w3, layer_b3):
    feats = [feat0, feat1, feat2, feat3]
    iw = [inner_w0, inner_w1, inner_w2, inner_w3]
    ib = [inner_b0, inner_b1, inner_b2, inner_b3]
    lw = [layer_w0, layer_w1, layer_w2, layer_w3]
    lb = [layer_b0, layer_b1, layer_b2, layer_b3]

    names = ["feat0", "feat1", "feat2", "feat3"]
    results = [None] * 4
    last_inner = None
    for idx in range(3, -1, -1):
        N, _, H, W = feats[idx].shape
        last_inner = _lateral(feats[idx], iw[idx], ib[idx], last_inner)
        results[idx] = _smooth(last_inner, lw[idx], lb[idx], N, H, W)

    from collections import OrderedDict
    return OrderedDict(zip(names, results))


# speedup vs baseline: 1.2033x; 1.0207x over previous
import jax
import jax.numpy as jnp
from jax.experimental import pallas as pl
from jax.experimental.pallas import tpu as pltpu


def _k1_body(x_ref, w_ref, b_ref, o_ref):
    x = x_ref[0].astype(jnp.bfloat16)
    y = jnp.dot(w_ref[...], x, preferred_element_type=jnp.float32)
    o_ref[0] = (y + b_ref[...]).astype(jnp.bfloat16)


def _k1_add_body(x_ref, w_ref, b_ref, s_ref, g_ref, o_ref):
    x = x_ref[0].astype(jnp.bfloat16)
    y = jnp.dot(w_ref[...], x, preferred_element_type=jnp.float32)
    up = jnp.dot(s_ref[0], g_ref[...],
                 preferred_element_type=jnp.float32)
    o_ref[0] = (y + b_ref[...] + up).astype(jnp.bfloat16)


def _upsample_gather(W, TS):
    jj = jnp.arange(TS)
    src = (jj // (2 * W)) * (W // 2) + (jj % W) // 2
    return (src[None, :] == jnp.arange(TS // 4)[:, None]).astype(jnp.bfloat16)


def _lateral(feat, w_oihw, bias, src_flat):
    N, Cin, H, W = feat.shape
    C = w_oihw.shape[0]
    HW = H * W
    x = feat.reshape(N, Cin, HW)
    w2 = w_oihw[:, :, 0, 0].astype(jnp.bfloat16)
    b2 = bias.reshape(C, 1).astype(jnp.float32)

    TS = HW if src_flat is None else min(2048, HW)
    grid = (N, HW // TS)
    in_specs = [
        pl.BlockSpec((1, Cin, TS), lambda n, j: (n, 0, j)),
        pl.BlockSpec((C, Cin), lambda n, j: (0, 0)),
        pl.BlockSpec((C, 1), lambda n, j: (0, 0)),
    ]
    args = [x, w2, b2]
    if src_flat is None:
        body = _k1_body
    else:
        body = _k1_add_body
        in_specs += [
            pl.BlockSpec((1, C, TS // 4), lambda n, j: (n, 0, j)),
            pl.BlockSpec((TS // 4, TS), lambda n, j: (0, 0)),
        ]
        args += [src_flat, _upsample_gather(W, TS)]

    bytes_acc = (N * HW * Cin * 4 + Cin * C * 2 + C * 4 + N * HW * C * 2
                 + (0 if src_flat is None else N * HW // 4 * C * 2))
    out = pl.pallas_call(
        body,
        out_shape=jax.ShapeDtypeStruct((N, C, HW), jnp.bfloat16),
        grid=grid,
        in_specs=in_specs,
        out_specs=pl.BlockSpec((1, C, TS), lambda n, j: (n, 0, j)),
        compiler_params=pltpu.CompilerParams(
            dimension_semantics=("parallel", "parallel"),
            vmem_limit_bytes=64 * 1024 * 1024,
        ),
        cost_estimate=pl.CostEstimate(
            flops=int(2 * N * HW * Cin * C), transcendentals=0,
            bytes_accessed=int(bytes_acc)),
    )(*args)
    return out


def _make_k2_body(TH, W, Ht, row_halo):
    S = TH * W

    def _body(xp_ref, xc_ref, xn_ref, w_ref, b_ref, o_ref):
        i = pl.program_id(1)
        C = xc_ref.shape[1]
        dt = xc_ref.dtype
        prev = xp_ref[0] if row_halo else xp_ref[0, :, S - W:]
        nxt = xn_ref[0] if row_halo else xn_ref[0, :, :W]
        top = jnp.where(i > 0, prev, jnp.zeros((C, W), dt))
        bot = jnp.where(i < Ht - 1, nxt, jnp.zeros((C, W), dt))
        xfull = jnp.concatenate([top, xc_ref[0], bot], axis=1)
        lane = jax.lax.broadcasted_iota(jnp.int32, (1, S + 2 * W), 1) % W
        zc = jnp.zeros((C, 1), dt)
        sL = jnp.concatenate([zc, xfull[:, :-1]], axis=1)
        sL = jnp.where(lane == 0, jnp.zeros((), dt), sL)
        sR = jnp.concatenate([xfull[:, 1:], zc], axis=1)
        sR = jnp.where(lane == W - 1, jnp.zeros((), dt), sR)
        srcs = (sL, xfull, sR)
        acc = None
        for dy in range(3):
            for dx in range(3):
                op = srcs[dx][:, dy * W: dy * W + S]
                d = jnp.dot(w_ref[3 * dy + dx], op,
                            preferred_element_type=jnp.float32)
                acc = d if acc is None else acc + d
        o_ref[0] = acc + b_ref[...]
    return _body


def _k2_row_tile(H, W):
    best = 1
    for th in range(1, H + 1):
        if H % th == 0 and th * W <= 4096 and H // th >= 2:
            best = th
    return best


def _smooth(inner_flat, w_oihw, bias, N, H, W):
    C = w_oihw.shape[0]
    TH = _k2_row_tile(H, W)
    Ht = H // TH
    w9 = jnp.transpose(w_oihw, (2, 3, 0, 1)).reshape(9, C, C)
    w9 = w9.astype(jnp.bfloat16)
    b2 = bias.reshape(C, 1).astype(jnp.float32)

    row_halo = (W % 128 == 0)
    if row_halo:
        in_specs = [
            pl.BlockSpec((1, C, W),
                         lambda n, i: (n, 0, jnp.clip(i * TH - 1, 0, H - 1))),
            pl.BlockSpec((1, C, TH * W), lambda n, i: (n, 0, i)),
            pl.BlockSpec((1, C, W),
                         lambda n, i: (n, 0, jnp.clip((i + 1) * TH, 0, H - 1))),
        ]
    else:
        in_specs = [
            pl.BlockSpec((1, C, TH * W),
                         lambda n, i, _k=k: (n, 0, jnp.clip(i + _k, 0, Ht - 1)))
            for k in (-1, 0, 1)
        ]
    in_specs += [
        pl.BlockSpec((9, C, C), lambda n, i: (0, 0, 0)),
        pl.BlockSpec((C, 1), lambda n, i: (0, 0)),
    ]
    flops = 2 * N * H * W * 9 * C * C
    bytes_acc = (N * (H + 2 * Ht) * W * C * 2 + 9 * C * C * 2 + C * 4
                 + N * H * W * C * 4)
    out = pl.pallas_call(
        _make_k2_body(TH, W, Ht, row_halo),
        out_shape=jax.ShapeDtypeStruct((N, C, H * W), jnp.float32),
        grid=(N, Ht),
        in_specs=in_specs,
        out_specs=pl.BlockSpec((1, C, TH * W), lambda n, i: (n, 0, i)),
        compiler_params=pltpu.CompilerParams(
            dimension_semantics=("parallel", "parallel"),
            vmem_limit_bytes=64 * 1024 * 1024,
        ),
        cost_estimate=pl.CostEstimate(
            flops=int(flops), transcendentals=0,
            bytes_accessed=int(bytes_acc)),
    )(inner_flat, inner_flat, inner_flat, w9, b2)
    return out.reshape(N, C, H, W)


def kernel(feat0, feat1, feat2, feat3,
           inner_w0, inner_b0, layer_w0, layer_b0,
           inner_w1, inner_b1, layer_w1, layer_b1,
           inner_w2, inner_b2, layer_w2, layer_b2,
           inner_w3, inner_b3, layer_w3, layer_b3):
    feats = [feat0, feat1, feat2, feat3]
    iw = [inner_w0, inner_w1, inner_w2, inner_w3]
    ib = [inner_b0, inner_b1, inner_b2, inner_b3]
    lw = [layer_w0, layer_w1, layer_w2, layer_w3]
    lb = [layer_b0, layer_b1, layer_b2, layer_b3]

    names = ["feat0", "feat1", "feat2", "feat3"]
    results = [None] * 4
    last_inner = None
    for idx in range(3, -1, -1):
        N, _, H, W = feats[idx].shape
        last_inner = _lateral(feats[idx], iw[idx], ib[idx], last_inner)
        results[idx] = _smooth(last_inner, lw[idx], lb[idx], N, H, W)

    from collections import OrderedDict
    return OrderedDict(zip(names, results))
